# baseline (speedup 1.0000x reference)
import functools

import jax
import jax.numpy as jnp
from jax import lax
from jax.experimental import pallas as pl
from jax.experimental.pallas import tpu as pltpu
from jax.experimental.pallas import tpu_sc as plsc

VOCAB = 19240
EMB = 10
BATCH = 4096
DP = 16
RB = 256
NSPLIT = 8
RSUB = RB // NSPLIT


def _make_sc_gather():
    info = plsc.get_sparse_core_info()
    nc, ns = info.num_cores, info.num_subcores
    nw = nc * ns
    bpw = BATCH // nw
    mesh = plsc.VectorSubcoreMesh(core_axis_name="c", subcore_axis_name="s")

    @functools.partial(
        pl.kernel,
        mesh=mesh,
        out_type=jax.ShapeDtypeStruct((BATCH, DP), jnp.float32),
        scratch_types=[
            pltpu.VMEM((bpw,), jnp.int32),
            pltpu.VMEM((bpw, DP), jnp.float32),
            pltpu.SemaphoreType.DMA,
        ],
        compiler_params=pltpu.CompilerParams(use_tc_tiling_on_sc=False),
    )
    def sc_gather(table_hbm, idx_hbm, out_hbm, idx_v, rows_v, sem):
        wid = lax.axis_index("s") * nc + lax.axis_index("c")
        base = wid * bpw
        pltpu.sync_copy(idx_hbm.at[pl.ds(base, bpw)], idx_v)
        pltpu.async_copy(table_hbm.at[idx_v], rows_v, sem).wait()
        pltpu.sync_copy(rows_v, out_hbm.at[pl.ds(base, bpw)])

    return sc_gather


def _tc_body(emb_ref, w1t_ref, w2t_ref, out_hbm, buf, sems):
    i = pl.program_id(0)
    n = pl.num_programs(0)
    slot = lax.rem(i, 2)

    def _copies(step, s):
        return [
            pltpu.make_async_copy(
                buf.at[s, pl.ds(j * RSUB, RSUB)],
                out_hbm.at[pl.ds(step * RB + j * RSUB, RSUB)],
                sems.at[s, j],
            )
            for j in range(NSPLIT)
        ]

    @pl.when(i >= 2)
    def _reclaim():
        for c in _copies(i - 2, slot):
            c.wait()

    emb = emb_ref[...]
    hidden = jnp.maximum(
        lax.dot_general(emb, w1t_ref[...], (((1,), (0,)), ((), ())),
                        preferred_element_type=jnp.float32),
        0.0,
    )
    buf[slot] = lax.dot_general(hidden, w2t_ref[...],
                                (((1,), (0,)), ((), ())),
                                preferred_element_type=jnp.float32)

    for c in _copies(i, slot):
        c.start()

    @pl.when(i == n - 1)
    def _drain():
        for c in _copies(i - 1, 1 - slot):
            c.wait()
        for c in _copies(i, slot):
            c.wait()


def _tc_mlp(emb, w1t, w2t):
    grid = (BATCH // RB,)
    return pl.pallas_call(
        _tc_body,
        grid=grid,
        in_specs=[
            pl.BlockSpec((RB, DP), lambda i: (i, 0)),
            pl.BlockSpec((DP, EMB), lambda i: (0, 0)),
            pl.BlockSpec((EMB, VOCAB), lambda i: (0, 0)),
        ],
        out_specs=pl.BlockSpec(memory_space=pltpu.MemorySpace.HBM),
        out_shape=jax.ShapeDtypeStruct((BATCH, VOCAB), jnp.float32),
        scratch_shapes=[
            pltpu.VMEM((2, RB, VOCAB), jnp.float32),
            pltpu.SemaphoreType.DMA((2, NSPLIT)),
        ],
    )(emb, w1t, w2t)


@jax.jit
def kernel(X, emb_table, W1, W2):
    X = X.astype(jnp.int32)
    table_p = jnp.pad(emb_table, ((0, 0), (0, DP - EMB)))
    w1t = jnp.pad(W1.T, ((0, DP - EMB), (0, 0)))
    emb = _make_sc_gather()(table_p, X)
    return _tc_mlp(emb, w1t, W2.T)


# speedup vs baseline: 3.1465x; 3.1465x over previous
import functools

import jax
import jax.numpy as jnp
from jax import lax
from jax.experimental import pallas as pl
from jax.experimental.pallas import tpu as pltpu
from jax.experimental.pallas import tpu_sc as plsc

VOCAB = 19240
EMB = 10
BATCH = 4096
DP = 128
VB = 512
NMAIN = VOCAB // VB
TAIL = VOCAB - NMAIN * VB
NSPLIT = 4
VSUB = VB // NSPLIT


def _make_sc_gather():
    info = plsc.get_sparse_core_info()
    nc, ns = info.num_cores, info.num_subcores
    nw = nc * ns
    bpw = BATCH // nw
    mesh = plsc.VectorSubcoreMesh(core_axis_name="c", subcore_axis_name="s")

    @functools.partial(
        pl.kernel,
        mesh=mesh,
        out_type=jax.ShapeDtypeStruct((BATCH, DP), jnp.float32),
        scratch_types=[
            pltpu.VMEM((bpw,), jnp.int32),
            pltpu.VMEM((bpw, DP), jnp.float32),
            pltpu.SemaphoreType.DMA,
        ],
    )
    def sc_gather(table_hbm, idx_hbm, out_hbm, idx_v, rows_v, sem):
        wid = lax.axis_index("s") * nc + lax.axis_index("c")
        base = wid * bpw
        pltpu.sync_copy(idx_hbm.at[pl.ds(base, bpw)], idx_v)
        pltpu.async_copy(table_hbm.at[idx_v], rows_v, sem).wait()
        pltpu.sync_copy(rows_v, out_hbm.at[pl.ds(base, bpw)])

    return sc_gather


def _tc_body(emb_ref, w1t_ref, w2t_ref, out_hbm, buf, hid, sems):
    i = pl.program_id(0)
    n = pl.num_programs(0)
    slot = lax.rem(i, 2)

    @pl.when(i == 0)
    def _prologue():
        hidden = jnp.maximum(
            lax.dot_general(emb_ref[...], w1t_ref[...], (((1,), (0,)), ((), ())),
                            preferred_element_type=jnp.float32),
            0.0,
        )
        hid[...] = hidden.T

    def _copies(step, s):
        return [
            pltpu.make_async_copy(
                buf.at[s, pl.ds(j * VSUB, VSUB)],
                out_hbm.at[pl.ds(step * VB + j * VSUB, VSUB)],
                sems.at[s, j],
            )
            for j in range(NSPLIT)
        ]

    def _tail_copy(s):
        return pltpu.make_async_copy(
            buf.at[s, pl.ds(0, TAIL)],
            out_hbm.at[pl.ds(NMAIN * VB, TAIL)],
            sems.at[s, NSPLIT],
        )

    @pl.when(i >= 2)
    def _reclaim():
        for c in _copies(i - 2, slot):
            c.wait()

    @pl.when(i < NMAIN)
    def _main():
        w2t_slice = w2t_ref[:, pl.ds(i * VB, VB)]
        buf[slot] = lax.dot_general(w2t_slice, hid[...],
                                    (((0,), (0,)), ((), ())),
                                    preferred_element_type=jnp.float32)
        for c in _copies(i, slot):
            c.start()

    @pl.when(i == NMAIN)
    def _tail():
        w2t_slice = w2t_ref[:, pl.ds(NMAIN * VB, TAIL)]
        buf[slot, pl.ds(0, TAIL)] = lax.dot_general(
            w2t_slice, hid[...], (((0,), (0,)), ((), ())),
            preferred_element_type=jnp.float32)
        _tail_copy(slot).start()

    @pl.when(i == n - 1)
    def _drain():
        for c in _copies(i - 1, 1 - slot):
            c.wait()
        _tail_copy(slot).wait()


def _tc_mlp(emb, w1t, w2t):
    grid = (NMAIN + 1,)
    return pl.pallas_call(
        _tc_body,
        grid=grid,
        in_specs=[
            pl.BlockSpec((BATCH, DP), lambda i: (0, 0)),
            pl.BlockSpec((DP, EMB), lambda i: (0, 0)),
            pl.BlockSpec((EMB, VOCAB), lambda i: (0, 0)),
        ],
        out_specs=pl.BlockSpec(memory_space=pltpu.MemorySpace.HBM),
        out_shape=jax.ShapeDtypeStruct((VOCAB, BATCH), jnp.float32),
        scratch_shapes=[
            pltpu.VMEM((2, VB, BATCH), jnp.float32),
            pltpu.VMEM((EMB, BATCH), jnp.float32),
            pltpu.SemaphoreType.DMA((2, NSPLIT + 1)),
        ],
    )(emb, w1t, w2t)


@jax.jit
def kernel(X, emb_table, W1, W2):
    X = X.astype(jnp.int32)
    table_p = jnp.pad(emb_table, ((0, 0), (0, DP - EMB)))
    w1t = jnp.pad(W1.T, ((0, DP - EMB), (0, 0)))
    emb = _make_sc_gather()(table_p, X)
    return _tc_mlp(emb, w1t, W2.T).T
